# baseline (speedup 1.0000x reference)
import functools

import jax
import jax.numpy as jnp
from jax import lax
from jax.experimental import pallas as pl
from jax.experimental.pallas import tpu as pltpu
from jax.experimental.pallas import tpu_sc as plsc

DELAY = 2048
NUM_ENVS = 16384

_NC = 1
_NS = 16
_NW = _NC * _NS
_BPW = NUM_ENVS // _NW
_L = 16
_CH = 128
_ND = _BPW // _CH


def _gather_body(i_hbm, buf_hbm, out_hbm, iv_v, *rest):
    idxc, (vals_v, sem) = rest[:_ND], rest[_ND:]
    wid = lax.axis_index("s") * _NC + lax.axis_index("c")
    base = wid * _BPW
    lane = lax.iota(jnp.int32, _L)

    pltpu.sync_copy(i_hbm.at[pl.ds(base, _BPW)], iv_v)

    cps = []
    for c in range(_ND):
        for k in range(_CH // _L):
            t = c * (_CH // _L) + k
            sl = pl.ds(t * _L, _L)
            iv16 = iv_v[sl]
            r16 = base + t * _L + lane
            w16 = (
                lax.shift_left(lax.shift_right_logical(r16, 3), 14)
                + lax.shift_left(lax.shift_right_logical(iv16, 7), 10)
                + lax.shift_left(r16 & 7, 7)
                + (iv16 & 127)
            )
            idxc[c][pl.ds(k * _L, _L)] = w16
        cps.append(
            pltpu.async_copy(
                buf_hbm.at[plsc.Indices(idxc[c])],
                vals_v.at[pl.ds(c * _CH, _CH)],
                sem,
            )
        )
    for c, cp in enumerate(cps):
        cp.wait()
        pltpu.sync_copy(
            vals_v.at[pl.ds(c * _CH, _CH)],
            out_hbm.at[pl.ds(base + c * _CH, _CH)],
        )


@functools.partial(
    pl.kernel,
    mesh=plsc.VectorSubcoreMesh(core_axis_name="c", subcore_axis_name="s", num_cores=1),
    out_type=jax.ShapeDtypeStruct((NUM_ENVS,), jnp.float32),
    scratch_types=[
        pltpu.VMEM((_BPW,), jnp.int32),
        *[pltpu.VMEM((_CH,), jnp.int32) for _ in range(_ND)],
        pltpu.VMEM((_BPW,), jnp.float32),
        pltpu.SemaphoreType.DMA,
    ],
)
def _sc_gather(i_hbm, buf_hbm, out_hbm, iv_v, *rest):
    _gather_body(i_hbm, buf_hbm, out_hbm, iv_v, *rest)


def kernel(x, buffor, i):
    del x
    flat = buffor.reshape(NUM_ENVS // 8, 8, DELAY // 128, 128)
    flat = flat.transpose(0, 2, 1, 3)
    flat = flat.reshape(NUM_ENVS * DELAY)
    return _sc_gather(i, flat)


# speedup vs baseline: 1.0079x; 1.0079x over previous
import functools

import jax
import jax.numpy as jnp
from jax import lax
from jax.experimental import pallas as pl
from jax.experimental.pallas import tpu as pltpu
from jax.experimental.pallas import tpu_sc as plsc

DELAY = 2048
NUM_ENVS = 16384

_NC = 1
_NS = 16
_NW = _NC * _NS
_BPW = NUM_ENVS // _NW
_L = 16
_CH = 128
_ND = _BPW // _CH


def _gather_body(i_hbm, buf_hbm, out_hbm, iv_v, *rest):
    idxc, (vals_v, sem) = rest[:_ND], rest[_ND:]
    wid = lax.axis_index("s") * _NC + lax.axis_index("c")
    base = wid * _BPW
    lane = lax.iota(jnp.int32, _L)

    pltpu.sync_copy(i_hbm.at[pl.ds(base, _BPW)], iv_v)

    cps = []
    for c in range(_ND):
        for k in range(_CH // _L):
            t = c * (_CH // _L) + k
            sl = pl.ds(t * _L, _L)
            iv16 = iv_v[sl]
            r16 = base + t * _L + lane
            w16 = (
                lax.shift_left(lax.shift_right_logical(r16, 3), 14)
                + lax.shift_left(lax.shift_right_logical(iv16, 7), 10)
                + lax.shift_left(r16 & 7, 7)
                + (iv16 & 127)
            )
            idxc[c][pl.ds(k * _L, _L)] = w16
        cps.append(
            pltpu.async_copy(
                buf_hbm.at[plsc.Indices(idxc[c])],
                vals_v.at[pl.ds(c * _CH, _CH)],
                sem,
            )
        )
    for cp in cps:
        cp.wait()

    pltpu.sync_copy(vals_v, out_hbm.at[pl.ds(base, _BPW)])


@functools.partial(
    pl.kernel,
    mesh=plsc.VectorSubcoreMesh(core_axis_name="c", subcore_axis_name="s", num_cores=1),
    out_type=jax.ShapeDtypeStruct((NUM_ENVS,), jnp.float32),
    scratch_types=[
        pltpu.VMEM((_BPW,), jnp.int32),
        *[pltpu.VMEM((_CH,), jnp.int32) for _ in range(_ND)],
        pltpu.VMEM((_BPW,), jnp.float32),
        pltpu.SemaphoreType.DMA,
    ],
)
def _sc_gather(i_hbm, buf_hbm, out_hbm, iv_v, *rest):
    _gather_body(i_hbm, buf_hbm, out_hbm, iv_v, *rest)


def kernel(x, buffor, i):
    del x
    flat = buffor.reshape(NUM_ENVS // 8, 8, DELAY // 128, 128)
    flat = flat.transpose(0, 2, 1, 3)
    flat = flat.reshape(NUM_ENVS * DELAY)
    return _sc_gather(i, flat)
